# Initial kernel scaffold; baseline (speedup 1.0000x reference)
import jax
import jax.numpy as jnp
from jax.experimental import pallas as pl


def kernel(feature, edge_index, edge_attr, se0, se1, se2, se3, w_time, b_time, W_ta, b_ta, Wc0, bc0, Wc1, bc1, Wc2, bc2, Wc3, bc3, W_out, b_out):
    raise NotImplementedError("write your pallas kernel here")


# speedup vs baseline: 12.5140x; 12.5140x over previous
import functools

import jax
import jax.numpy as jnp
from jax import lax
from jax.experimental import pallas as pl
from jax.experimental.pallas import tpu as pltpu
from jax.experimental.pallas import tpu_sc as plsc

_N = 10000
_E = 320000
_D = 128
_H = 64
_NB = 4
_NC = 40

_EPAD = 327680
_CH = _EPAD // 128
_CH_TE = _CH // 32
_CH_BD = _CH // 16
_NP = 10240
_RPT = _NP // 16

_BE = 4096
_BR = 2048


def _sc_mesh():
    return plsc.VectorSubcoreMesh(core_axis_name="c", subcore_axis_name="s")


def _te_body(t_ref, w_ref, b_ref, o_ref):
    o_ref[...] = jnp.cos(t_ref[...] * w_ref[...] + b_ref[...])


def _tc_time_encode(t_pad, wt, bt):
    return pl.pallas_call(
        _te_body,
        grid=(_EPAD // _BE,),
        in_specs=[pl.BlockSpec((_BE, 1), lambda i: (i, 0)),
                  pl.BlockSpec((1, _H), lambda i: (0, 0)),
                  pl.BlockSpec((1, _H), lambda i: (0, 0))],
        out_specs=pl.BlockSpec((_BE, _H), lambda i: (i, 0)),
        out_shape=jax.ShapeDtypeStruct((_EPAD, _H), jnp.float32),
    )(t_pad, wt, bt)


@functools.partial(
    pl.kernel,
    out_type=(
        jax.ShapeDtypeStruct((2, _NP, _H), jnp.float32),
        jax.ShapeDtypeStruct((_NB, _NP, 16), jnp.float32),
    ),
    mesh=_sc_mesh(),
    scratch_types=(
        pltpu.VMEM((_CH_BD, 128), jnp.int32),
        pltpu.VMEM((256, _H), jnp.float32),
        pltpu.VMEM((128, 16), jnp.float32),
        pltpu.VMEM_SHARED((_NP, _H), jnp.float32),
        pltpu.VMEM_SHARED((_NP, 16), jnp.float32),
    ),
    compiler_params=pltpu.CompilerParams(use_tc_tiling_on_sc=False),
)
def _sc_agg_deg(te, dst, c0, c1, c2, c3, zrows, zdeg, ones_hbm, aggp, degc,
                idx_v, row_v, ones_v, acc_sh, dacc_sh):
    c = lax.axis_index("c")
    s = lax.axis_index("s")
    wid = c * 16 + s
    pltpu.sync_copy(zrows.at[pl.ds(0, 256)], row_v)
    for q, sz in ((0, 256), (256, 256), (512, 128)):
        pltpu.sync_copy(row_v.at[pl.ds(0, sz)],
                        acc_sh.at[pl.ds(s * _RPT + q, sz)])
    plsc.subcore_barrier()
    pltpu.sync_copy(dst.at[pl.ds(wid * _CH_TE, _CH_TE)],
                    idx_v.at[pl.ds(0, _CH_TE)])

    def te_group(g, carry):
        base = (wid * _CH_TE + g * 2) * 128
        pltpu.sync_copy(te.at[pl.ds(base, 256)], row_v)
        for k in range(2):
            pltpu.sync_copy(row_v.at[pl.ds(k * 128, 128)],
                            acc_sh.at[idx_v.at[g * 2 + k]], add=True)
        return carry

    lax.fori_loop(0, _CH_TE // 2, te_group, 0)
    plsc.subcore_barrier()
    for q, sz in ((0, 256), (256, 256), (512, 128)):
        pltpu.sync_copy(acc_sh.at[pl.ds(s * _RPT + q, sz)],
                        row_v.at[pl.ds(0, sz)])
        pltpu.sync_copy(row_v.at[pl.ds(0, sz)],
                        aggp.at[c, pl.ds(s * _RPT + q, sz)])
    cols = [c0, c1, c2, c3]
    for b in range(_NB):
        @pl.when(c == b // 2)
        def _band():
            pltpu.sync_copy(zdeg, ones_v)
            for q in range(_RPT // 128):
                pltpu.sync_copy(ones_v,
                                dacc_sh.at[pl.ds(s * _RPT + q * 128, 128)])
            plsc.subcore_barrier()
            pltpu.sync_copy(ones_hbm, ones_v)
            pltpu.sync_copy(cols[b].at[pl.ds(s * _CH_BD, _CH_BD)], idx_v)

            def hist_step(j, carry):
                pltpu.sync_copy(ones_v, dacc_sh.at[idx_v.at[j]], add=True)
                return carry

            lax.fori_loop(0, _CH_BD, hist_step, 0)
            plsc.subcore_barrier()
            for q in range(_RPT // 128):
                pltpu.sync_copy(dacc_sh.at[pl.ds(s * _RPT + q * 128, 128)],
                                ones_v)
                pltpu.sync_copy(ones_v,
                                degc.at[b, pl.ds(s * _RPT + q * 128, 128)])
            plsc.subcore_barrier()


def _mid_body(aggp, degc, f, Wta, bta, Wf, Wt,
              y0, y1, y2, y3, d0, d1, d2, d3):
    agg = aggp[0] + aggp[1]
    ta = jnp.dot(agg, Wta[...], preferred_element_type=jnp.float32) + bta[...]
    xw = (jnp.dot(f[...], Wf[...], preferred_element_type=jnp.float32)
          + jnp.dot(ta, Wt[...], preferred_element_type=jnp.float32))
    ys = [y0, y1, y2, y3]
    ds = [d0, d1, d2, d3]
    for b in range(_NB):
        deg = degc[b][:, 0:1] + 1.0
        dinv = lax.rsqrt(deg)
        ds[b][...] = dinv
        ys[b][...] = dinv * xw[:, b * _H:(b + 1) * _H]


def _tc_mid(aggp, degc, f, Wta, bta, Wf, Wt):
    ys = [jax.ShapeDtypeStruct((_NP, _H), jnp.float32) for _ in range(_NB)]
    dvs = [jax.ShapeDtypeStruct((_NP, 1), jnp.float32) for _ in range(_NB)]
    return pl.pallas_call(
        _mid_body,
        grid=(_NP // _BR,),
        in_specs=[
            pl.BlockSpec((2, _BR, _H), lambda i: (0, i, 0)),
            pl.BlockSpec((_NB, _BR, 16), lambda i: (0, i, 0)),
            pl.BlockSpec((_BR, _D), lambda i: (i, 0)),
            pl.BlockSpec((_H, _H), lambda i: (0, 0)),
            pl.BlockSpec((1, _H), lambda i: (0, 0)),
            pl.BlockSpec((_D, _NB * _H), lambda i: (0, 0)),
            pl.BlockSpec((_H, _NB * _H), lambda i: (0, 0)),
        ],
        out_specs=[pl.BlockSpec((_BR, _H), lambda i: (i, 0))] * _NB
        + [pl.BlockSpec((_BR, 1), lambda i: (i, 0))] * _NB,
        out_shape=ys + dvs,
    )(aggp, degc, f, Wta, bta, Wf, Wt)


@functools.partial(
    pl.kernel,
    out_type=jax.ShapeDtypeStruct((_NB, _NP, _H), jnp.float32),
    mesh=_sc_mesh(),
    scratch_types=(
        pltpu.VMEM((_CH_BD, 128), jnp.int32),
        pltpu.VMEM((_CH_BD, 128), jnp.int32),
        pltpu.VMEM((128, _H), jnp.float32),
        pltpu.VMEM_SHARED((_NP, _H), jnp.float32),
        pltpu.SemaphoreType.DMA,
    ),
    compiler_params=pltpu.CompilerParams(use_tc_tiling_on_sc=False),
)
def _sc_conv(y0, y1, y2, y3, r0, r1, r2, r3, k0, k1, k2, k3, zrows, scat,
             rows_v, cols_v, gbuf, acc_sh, sem):
    cid = lax.axis_index("c")
    s = lax.axis_index("s")
    ys = [y0, y1, y2, y3]
    rs = [r0, r1, r2, r3]
    cs = [k0, k1, k2, k3]
    for b in range(_NB):
        @pl.when(cid == b // 2)
        def _band():
            pltpu.sync_copy(zrows.at[pl.ds(0, 128)], gbuf)
            for q in range(_RPT // 128):
                pltpu.sync_copy(gbuf,
                                acc_sh.at[pl.ds(s * _RPT + q * 128, 128)])
            plsc.subcore_barrier()
            pltpu.sync_copy(rs[b].at[pl.ds(s * _CH_BD, _CH_BD)], rows_v)
            pltpu.sync_copy(cs[b].at[pl.ds(s * _CH_BD, _CH_BD)], cols_v)

            def step(j, carry):
                pltpu.async_copy(ys[b].at[rows_v.at[j]], gbuf, sem).wait()
                pltpu.sync_copy(gbuf, acc_sh.at[cols_v.at[j]], add=True)
                return carry

            lax.fori_loop(0, _CH_BD, step, 0)
            plsc.subcore_barrier()
            for q in range(_RPT // 128):
                pltpu.sync_copy(acc_sh.at[pl.ds(s * _RPT + q * 128, 128)],
                                gbuf)
                pltpu.sync_copy(gbuf,
                                scat.at[b, pl.ds(s * _RPT + q * 128, 128)])
            plsc.subcore_barrier()


def _fin_body(scat, y0, y1, y2, y3, d0, d1, d2, d3,
              bc0, bc1, bc2, bc3, Wout, bout, o_ref):
    ys = [y0, y1, y2, y3]
    ds = [d0, d1, d2, d3]
    bcs = [bc0, bc1, bc2, bc3]
    parts = []
    for b in range(_NB):
        h = ds[b][...] * (scat[b] + ys[b][...]) + bcs[b][...]
        parts.append(jnp.maximum(h, 0.0))
    xc = jnp.concatenate(parts, axis=1)
    logits = jnp.dot(xc, Wout[...], preferred_element_type=jnp.float32) + bout[...]
    m = jnp.max(logits, axis=1, keepdims=True)
    lse = jnp.log(jnp.sum(jnp.exp(logits - m), axis=1, keepdims=True)) + m
    o_ref[...] = logits - lse


def _tc_final(scat, ys, dvs, bcs, Wout, bout):
    return pl.pallas_call(
        _fin_body,
        grid=(_NP // _BR,),
        in_specs=[pl.BlockSpec((_NB, _BR, _H), lambda i: (0, i, 0))]
        + [pl.BlockSpec((_BR, _H), lambda i: (i, 0))] * _NB
        + [pl.BlockSpec((_BR, 1), lambda i: (i, 0))] * _NB
        + [pl.BlockSpec((1, _H), lambda i: (0, 0))] * _NB
        + [pl.BlockSpec((_NB * _H, _NC), lambda i: (0, 0)),
           pl.BlockSpec((1, _NC), lambda i: (0, 0))],
        out_specs=pl.BlockSpec((_BR, _NC), lambda i: (i, 0)),
        out_shape=jax.ShapeDtypeStruct((_NP, _NC), jnp.float32),
    )(scat, *ys, *dvs, *bcs, Wout, bout)


def kernel(feature, edge_index, edge_attr, se0, se1, se2, se3, w_time, b_time,
           W_ta, b_ta, Wc0, bc0, Wc1, bc1, Wc2, bc2, Wc3, bc3, W_out, b_out):
    n = _N
    pad = _EPAD - _E
    t_pad = jnp.pad(edge_attr[:, :1], ((0, pad), (0, 0)))
    dst2d = jnp.pad(edge_index[1], (0, pad),
                    constant_values=n).reshape(_CH, 128)
    rows2d, cols2d = [], []
    for se in (se0, se1, se2, se3):
        rows2d.append(jnp.pad(se[0], (0, pad)).reshape(_CH, 128))
        cols2d.append(jnp.pad(se[1], (0, pad),
                              constant_values=n).reshape(_CH, 128))
    zrows = jnp.zeros((256, _H), jnp.float32)
    zdeg = jnp.zeros((128, 16), jnp.float32)
    ones_hbm = jnp.ones((128, 16), jnp.float32)
    Wf = jnp.concatenate([Wc0[:_D], Wc1[:_D], Wc2[:_D], Wc3[:_D]], axis=1)
    Wt = jnp.concatenate([Wc0[_D:], Wc1[_D:], Wc2[_D:], Wc3[_D:]], axis=1)
    feat_p = jnp.pad(feature, ((0, _NP - n), (0, 0)))

    te = _tc_time_encode(t_pad, w_time.T, b_time.reshape(1, _H))
    aggp, degc = _sc_agg_deg(te, dst2d, *cols2d, zrows, zdeg, ones_hbm)
    mid = _tc_mid(aggp, degc, feat_p, W_ta, b_ta.reshape(1, _H), Wf, Wt)
    ys, dvs = list(mid[:_NB]), list(mid[_NB:])
    scat = _sc_conv(*ys, *rows2d, *cols2d, zrows)
    bcs = [bc0.reshape(1, _H), bc1.reshape(1, _H),
           bc2.reshape(1, _H), bc3.reshape(1, _H)]
    out = _tc_final(scat, ys, dvs, bcs, W_out, b_out.reshape(1, _NC))
    return out[:n]


# speedup vs baseline: 14.8562x; 1.1872x over previous
import functools

import jax
import jax.numpy as jnp
from jax import lax
from jax.experimental import pallas as pl
from jax.experimental.pallas import tpu as pltpu
from jax.experimental.pallas import tpu_sc as plsc

_N = 10000
_E = 320000
_D = 128
_H = 64
_NB = 4
_NC = 40

_EPAD = 327680
_CH = _EPAD // 128
_CH_TE = _CH // 32
_CH_BD = _CH // 16
_NP = 10240
_RPT = _NP // 16

_BE = 4096
_BR = 2048


def _sc_mesh():
    return plsc.VectorSubcoreMesh(core_axis_name="c", subcore_axis_name="s")


def _te_body(t_ref, w_ref, b_ref, o_ref):
    o_ref[...] = jnp.cos(t_ref[...] * w_ref[...] + b_ref[...])


def _tc_time_encode(t_pad, wt, bt):
    return pl.pallas_call(
        _te_body,
        grid=(_EPAD // _BE,),
        in_specs=[pl.BlockSpec((_BE, 1), lambda i: (i, 0)),
                  pl.BlockSpec((1, _H), lambda i: (0, 0)),
                  pl.BlockSpec((1, _H), lambda i: (0, 0))],
        out_specs=pl.BlockSpec((_BE, _H), lambda i: (i, 0)),
        out_shape=jax.ShapeDtypeStruct((_EPAD, _H), jnp.float32),
    )(t_pad, wt, bt)


@functools.partial(
    pl.kernel,
    out_type=(
        jax.ShapeDtypeStruct((2, _NP, _H), jnp.float32),
        jax.ShapeDtypeStruct((_NB, _NP, 16), jnp.float32),
    ),
    mesh=_sc_mesh(),
    scratch_types=(
        pltpu.VMEM((_CH_BD, 128), jnp.int32),
        pltpu.VMEM((256, _H), jnp.float32),
        pltpu.VMEM((128, 16), jnp.float32),
        pltpu.VMEM_SHARED((_NP, _H), jnp.float32),
        pltpu.VMEM_SHARED((_NP, 16), jnp.float32),
    ),
    compiler_params=pltpu.CompilerParams(use_tc_tiling_on_sc=False),
)
def _sc_agg_deg(te, dst, c0, c1, c2, c3, zrows, zdeg, ones_hbm, aggp, degc,
                idx_v, row_v, ones_v, acc_sh, dacc_sh):
    c = lax.axis_index("c")
    s = lax.axis_index("s")
    wid = c * 16 + s
    pltpu.sync_copy(zrows.at[pl.ds(0, 256)], row_v)
    for q, sz in ((0, 256), (256, 256), (512, 128)):
        pltpu.sync_copy(row_v.at[pl.ds(0, sz)],
                        acc_sh.at[pl.ds(s * _RPT + q, sz)])
    plsc.subcore_barrier()
    pltpu.sync_copy(dst.at[pl.ds(wid * _CH_TE, _CH_TE)],
                    idx_v.at[pl.ds(0, _CH_TE)])

    def te_group(g, carry):
        base = (wid * _CH_TE + g * 2) * 128
        pltpu.sync_copy(te.at[pl.ds(base, 256)], row_v)
        for k in range(2):
            pltpu.sync_copy(row_v.at[pl.ds(k * 128, 128)],
                            acc_sh.at[idx_v.at[g * 2 + k]], add=True)
        return carry

    lax.fori_loop(0, _CH_TE // 2, te_group, 0)
    plsc.subcore_barrier()
    for q, sz in ((0, 256), (256, 256), (512, 128)):
        pltpu.sync_copy(acc_sh.at[pl.ds(s * _RPT + q, sz)],
                        row_v.at[pl.ds(0, sz)])
        pltpu.sync_copy(row_v.at[pl.ds(0, sz)],
                        aggp.at[c, pl.ds(s * _RPT + q, sz)])
    cols = [c0, c1, c2, c3]
    for b in range(_NB):
        @pl.when(c == b // 2)
        def _band():
            pltpu.sync_copy(zdeg, ones_v)
            for q in range(_RPT // 128):
                pltpu.sync_copy(ones_v,
                                dacc_sh.at[pl.ds(s * _RPT + q * 128, 128)])
            plsc.subcore_barrier()
            pltpu.sync_copy(ones_hbm, ones_v)
            pltpu.sync_copy(cols[b].at[pl.ds(s * _CH_BD, _CH_BD)], idx_v)

            def hist_step(j, carry):
                pltpu.sync_copy(ones_v, dacc_sh.at[idx_v.at[j]], add=True)
                return carry

            lax.fori_loop(0, _CH_BD, hist_step, 0)
            plsc.subcore_barrier()
            for q in range(_RPT // 128):
                pltpu.sync_copy(dacc_sh.at[pl.ds(s * _RPT + q * 128, 128)],
                                ones_v)
                pltpu.sync_copy(ones_v,
                                degc.at[b, pl.ds(s * _RPT + q * 128, 128)])
            plsc.subcore_barrier()


def _mid_body(aggp, degc, f, Wta, bta, Wf, Wt,
              y0, y1, y2, y3, d0, d1, d2, d3):
    agg = aggp[0] + aggp[1]
    ta = jnp.dot(agg, Wta[...], preferred_element_type=jnp.float32) + bta[...]
    xw = (jnp.dot(f[...], Wf[...], preferred_element_type=jnp.float32)
          + jnp.dot(ta, Wt[...], preferred_element_type=jnp.float32))
    ys = [y0, y1, y2, y3]
    ds = [d0, d1, d2, d3]
    for b in range(_NB):
        deg = degc[b][:, 0:1] + 1.0
        dinv = lax.rsqrt(deg)
        ds[b][...] = dinv
        ys[b][...] = dinv * xw[:, b * _H:(b + 1) * _H]


def _tc_mid(aggp, degc, f, Wta, bta, Wf, Wt):
    ys = [jax.ShapeDtypeStruct((_NP, _H), jnp.float32) for _ in range(_NB)]
    dvs = [jax.ShapeDtypeStruct((_NP, 1), jnp.float32) for _ in range(_NB)]
    return pl.pallas_call(
        _mid_body,
        grid=(_NP // _BR,),
        in_specs=[
            pl.BlockSpec((2, _BR, _H), lambda i: (0, i, 0)),
            pl.BlockSpec((_NB, _BR, 16), lambda i: (0, i, 0)),
            pl.BlockSpec((_BR, _D), lambda i: (i, 0)),
            pl.BlockSpec((_H, _H), lambda i: (0, 0)),
            pl.BlockSpec((1, _H), lambda i: (0, 0)),
            pl.BlockSpec((_D, _NB * _H), lambda i: (0, 0)),
            pl.BlockSpec((_H, _NB * _H), lambda i: (0, 0)),
        ],
        out_specs=[pl.BlockSpec((_BR, _H), lambda i: (i, 0))] * _NB
        + [pl.BlockSpec((_BR, 1), lambda i: (i, 0))] * _NB,
        out_shape=ys + dvs,
    )(aggp, degc, f, Wta, bta, Wf, Wt)


_CH_HALF = _CH_BD // 2


@functools.partial(
    pl.kernel,
    out_type=jax.ShapeDtypeStruct((_NB, _NP, _H), jnp.float32),
    mesh=_sc_mesh(),
    scratch_types=(
        pltpu.VMEM((_CH_HALF, 128), jnp.int32),
        pltpu.VMEM((_CH_HALF, 128), jnp.int32),
        pltpu.VMEM((128, _H), jnp.float32),
        pltpu.VMEM_SHARED((_NP, _H), jnp.float32),
        pltpu.VMEM_SHARED((_NP, _H), jnp.float32),
    ),
    compiler_params=pltpu.CompilerParams(use_tc_tiling_on_sc=False),
)
def _sc_conv(y0, y1, y2, y3, r0, r1, r2, r3, k0, k1, k2, k3, zrows, scat,
             rows_v, cols_v, gbuf, y_sh, acc_sh):
    cid = lax.axis_index("c")
    s = lax.axis_index("s")
    ys = [y0, y1, y2, y3]
    rs = [r0, r1, r2, r3]
    cs = [k0, k1, k2, k3]
    for b in range(_NB):
        @pl.when(cid == b // 2)
        def _band():
            pltpu.sync_copy(zrows.at[pl.ds(0, 128)], gbuf)
            for q in range(_RPT // 128):
                pltpu.sync_copy(gbuf,
                                acc_sh.at[pl.ds(s * _RPT + q * 128, 128)])
            for q in range(_RPT // 128):
                pltpu.sync_copy(ys[b].at[pl.ds(s * _RPT + q * 128, 128)],
                                gbuf)
                pltpu.sync_copy(gbuf,
                                y_sh.at[pl.ds(s * _RPT + q * 128, 128)])
            plsc.subcore_barrier()
            for h in range(2):
                pltpu.sync_copy(
                    rs[b].at[pl.ds(s * _CH_BD + h * _CH_HALF, _CH_HALF)],
                    rows_v)
                pltpu.sync_copy(
                    cs[b].at[pl.ds(s * _CH_BD + h * _CH_HALF, _CH_HALF)],
                    cols_v)

                def step(j, carry):
                    pltpu.sync_copy(y_sh.at[rows_v.at[j]], gbuf)
                    pltpu.sync_copy(gbuf, acc_sh.at[cols_v.at[j]], add=True)
                    return carry

                lax.fori_loop(0, _CH_HALF, step, 0)
            plsc.subcore_barrier()
            for q in range(_RPT // 128):
                pltpu.sync_copy(acc_sh.at[pl.ds(s * _RPT + q * 128, 128)],
                                gbuf)
                pltpu.sync_copy(gbuf,
                                scat.at[b, pl.ds(s * _RPT + q * 128, 128)])
            plsc.subcore_barrier()


def _fin_body(scat, y0, y1, y2, y3, d0, d1, d2, d3,
              bc0, bc1, bc2, bc3, Wout, bout, o_ref):
    ys = [y0, y1, y2, y3]
    ds = [d0, d1, d2, d3]
    bcs = [bc0, bc1, bc2, bc3]
    parts = []
    for b in range(_NB):
        h = ds[b][...] * (scat[b] + ys[b][...]) + bcs[b][...]
        parts.append(jnp.maximum(h, 0.0))
    xc = jnp.concatenate(parts, axis=1)
    logits = jnp.dot(xc, Wout[...], preferred_element_type=jnp.float32) + bout[...]
    m = jnp.max(logits, axis=1, keepdims=True)
    lse = jnp.log(jnp.sum(jnp.exp(logits - m), axis=1, keepdims=True)) + m
    o_ref[...] = logits - lse


def _tc_final(scat, ys, dvs, bcs, Wout, bout):
    return pl.pallas_call(
        _fin_body,
        grid=(_NP // _BR,),
        in_specs=[pl.BlockSpec((_NB, _BR, _H), lambda i: (0, i, 0))]
        + [pl.BlockSpec((_BR, _H), lambda i: (i, 0))] * _NB
        + [pl.BlockSpec((_BR, 1), lambda i: (i, 0))] * _NB
        + [pl.BlockSpec((1, _H), lambda i: (0, 0))] * _NB
        + [pl.BlockSpec((_NB * _H, _NC), lambda i: (0, 0)),
           pl.BlockSpec((1, _NC), lambda i: (0, 0))],
        out_specs=pl.BlockSpec((_BR, _NC), lambda i: (i, 0)),
        out_shape=jax.ShapeDtypeStruct((_NP, _NC), jnp.float32),
    )(scat, *ys, *dvs, *bcs, Wout, bout)


def kernel(feature, edge_index, edge_attr, se0, se1, se2, se3, w_time, b_time,
           W_ta, b_ta, Wc0, bc0, Wc1, bc1, Wc2, bc2, Wc3, bc3, W_out, b_out):
    n = _N
    pad = _EPAD - _E
    t_pad = jnp.pad(edge_attr[:, :1], ((0, pad), (0, 0)))
    dst2d = jnp.pad(edge_index[1], (0, pad),
                    constant_values=n).reshape(_CH, 128)
    rows2d, cols2d = [], []
    for se in (se0, se1, se2, se3):
        rows2d.append(jnp.pad(se[0], (0, pad)).reshape(_CH, 128))
        cols2d.append(jnp.pad(se[1], (0, pad),
                              constant_values=n).reshape(_CH, 128))
    zrows = jnp.zeros((256, _H), jnp.float32)
    zdeg = jnp.zeros((128, 16), jnp.float32)
    ones_hbm = jnp.ones((128, 16), jnp.float32)
    Wf = jnp.concatenate([Wc0[:_D], Wc1[:_D], Wc2[:_D], Wc3[:_D]], axis=1)
    Wt = jnp.concatenate([Wc0[_D:], Wc1[_D:], Wc2[_D:], Wc3[_D:]], axis=1)
    feat_p = jnp.pad(feature, ((0, _NP - n), (0, 0)))

    te = _tc_time_encode(t_pad, w_time.T, b_time.reshape(1, _H))
    aggp, degc = _sc_agg_deg(te, dst2d, *cols2d, zrows, zdeg, ones_hbm)
    mid = _tc_mid(aggp, degc, feat_p, W_ta, b_ta.reshape(1, _H), Wf, Wt)
    ys, dvs = list(mid[:_NB]), list(mid[_NB:])
    scat = _sc_conv(*ys, *rows2d, *cols2d, zrows)
    bcs = [bc0.reshape(1, _H), bc1.reshape(1, _H),
           bc2.reshape(1, _H), bc3.reshape(1, _H)]
    out = _tc_final(scat, ys, dvs, bcs, W_out, b_out.reshape(1, _NC))
    return out[:n]
